# baseline (speedup 1.0000x reference)
import jax
import jax.numpy as jnp
from jax import lax
from jax.experimental import pallas as pl
from jax.experimental.pallas import tpu as pltpu
from jax.experimental.pallas import tpu_sc as plsc

_NUM_CORES = 2
_NUM_SUBCORES = 16
_NUM_WORKERS = _NUM_CORES * _NUM_SUBCORES

_VOCAB_TILE = 640


def _sc_gather_rows(table, idx):
    n = idx.shape[0]
    d = table.shape[1]
    b_per_w = n // _NUM_WORKERS
    mesh = plsc.VectorSubcoreMesh(core_axis_name="c", subcore_axis_name="s")

    @pl.kernel(
        out_type=jax.ShapeDtypeStruct((n, d), table.dtype),
        mesh=mesh,
        scratch_types=[
            pltpu.VMEM((b_per_w,), jnp.int32),
            pltpu.VMEM((b_per_w, d), table.dtype),
            pltpu.SemaphoreType.DMA,
        ],
    )
    def gather_kernel(tab_hbm, idx_hbm, out_hbm, idx_v, rows_v, sem):
        wid = lax.axis_index("s") * _NUM_CORES + lax.axis_index("c")
        base = wid * b_per_w
        pltpu.sync_copy(idx_hbm.at[pl.ds(base, b_per_w)], idx_v)
        pltpu.async_copy(tab_hbm.at[idx_v], rows_v, sem).wait()
        pltpu.sync_copy(rows_v, out_hbm.at[pl.ds(base, b_per_w)])

    return gather_kernel(table, idx)


def _tc_decode(x, w, bias2d, seq, batch):
    k = x.shape[1]
    v = w.shape[0]

    def mm(x_ref, w_ref, b_ref, o_ref, xbf_ref):
        @pl.when(pl.program_id(0) == 0)
        def _():
            xbf_ref[...] = x_ref[...].astype(jnp.bfloat16)

        d = jax.lax.dot_general(
            xbf_ref[...],
            w_ref[...].astype(jnp.bfloat16),
            (((1,), (1,)), ((), ())),
            preferred_element_type=jnp.float32,
        ) + b_ref[...]
        o_ref[...] = d.reshape(seq, batch, _VOCAB_TILE)

    return pl.pallas_call(
        mm,
        grid=(v // _VOCAB_TILE,),
        in_specs=[
            pl.BlockSpec((seq * batch, k), lambda j: (0, 0)),
            pl.BlockSpec((_VOCAB_TILE, k), lambda j: (j, 0)),
            pl.BlockSpec((1, _VOCAB_TILE), lambda j: (0, j)),
        ],
        out_specs=pl.BlockSpec((seq, batch, _VOCAB_TILE), lambda j: (0, 0, j)),
        out_shape=jax.ShapeDtypeStruct((seq, batch, v), jnp.float32),
        scratch_shapes=[pltpu.VMEM((seq * batch, k), jnp.bfloat16)],
    )(x, w, bias2d)


def kernel(src, emb, bias):
    seq, batch = src.shape
    vocab, dim = emb.shape
    n = seq * batch
    idx = src.reshape(n).astype(jnp.int32)
    x = _sc_gather_rows(emb, idx)
    return _tc_decode(x, emb, bias.reshape(1, vocab), seq, batch)


# speedup vs baseline: 1.1546x; 1.1546x over previous
import jax
import jax.numpy as jnp
from jax import lax
from jax.experimental import pallas as pl
from jax.experimental.pallas import tpu as pltpu
from jax.experimental.pallas import tpu_sc as plsc

_NUM_CORES = 2
_NUM_SUBCORES = 16
_NUM_WORKERS = _NUM_CORES * _NUM_SUBCORES

_VOCAB_TILE = 1280


def _sc_gather_rows(table, idx):
    n = idx.shape[0]
    d = table.shape[1]
    b_per_w = n // _NUM_WORKERS
    mesh = plsc.VectorSubcoreMesh(core_axis_name="c", subcore_axis_name="s")

    half = b_per_w // 2

    @pl.kernel(
        out_type=jax.ShapeDtypeStruct((n, d), table.dtype),
        mesh=mesh,
        scratch_types=[
            pltpu.VMEM((half,), jnp.int32),
            pltpu.VMEM((half,), jnp.int32),
            pltpu.VMEM((half, d), table.dtype),
            pltpu.VMEM((half, d), table.dtype),
            pltpu.SemaphoreType.DMA,
            pltpu.SemaphoreType.DMA,
            pltpu.SemaphoreType.DMA,
        ],
    )
    def gather_kernel(tab_hbm, idx_hbm, out_hbm,
                      idx_a, idx_b, rows_a, rows_b, sem_a, sem_b, sem_w):
        wid = lax.axis_index("s") * _NUM_CORES + lax.axis_index("c")
        base = wid * b_per_w
        pltpu.sync_copy(idx_hbm.at[pl.ds(base, half)], idx_a)
        ga = pltpu.async_copy(tab_hbm.at[idx_a], rows_a, sem_a)
        pltpu.sync_copy(idx_hbm.at[pl.ds(base + half, half)], idx_b)
        gb = pltpu.async_copy(tab_hbm.at[idx_b], rows_b, sem_b)
        ga.wait()
        wa = pltpu.async_copy(rows_a, out_hbm.at[pl.ds(base, half)], sem_w)
        gb.wait()
        pltpu.sync_copy(rows_b, out_hbm.at[pl.ds(base + half, half)])
        wa.wait()

    return gather_kernel(table, idx)


def _tc_decode(x, w, bias2d, seq, batch):
    k = x.shape[1]
    v = w.shape[0]

    def mm(x_ref, w_ref, b_ref, o_ref, xbf_ref):
        @pl.when(pl.program_id(0) == 0)
        def _():
            xbf_ref[...] = x_ref[...].astype(jnp.bfloat16)

        d = jax.lax.dot_general(
            xbf_ref[...],
            w_ref[...].astype(jnp.bfloat16),
            (((1,), (1,)), ((), ())),
            preferred_element_type=jnp.float32,
        ) + b_ref[...]
        o_ref[...] = d.reshape(seq, batch, _VOCAB_TILE)

    return pl.pallas_call(
        mm,
        grid=(v // _VOCAB_TILE,),
        in_specs=[
            pl.BlockSpec((seq * batch, k), lambda j: (0, 0)),
            pl.BlockSpec((_VOCAB_TILE, k), lambda j: (j, 0)),
            pl.BlockSpec((1, _VOCAB_TILE), lambda j: (0, j)),
        ],
        out_specs=pl.BlockSpec((seq, batch, _VOCAB_TILE), lambda j: (0, 0, j)),
        out_shape=jax.ShapeDtypeStruct((seq, batch, v), jnp.float32),
        scratch_shapes=[pltpu.VMEM((seq * batch, k), jnp.bfloat16)],
    )(x, w, bias2d)


def kernel(src, emb, bias):
    seq, batch = src.shape
    vocab, dim = emb.shape
    n = seq * batch
    idx = src.reshape(n).astype(jnp.int32)
    x = _sc_gather_rows(emb, idx)
    return _tc_decode(x, emb, bias.reshape(1, vocab), seq, batch)
